# Initial kernel scaffold; baseline (speedup 1.0000x reference)
import jax
import jax.numpy as jnp
from jax.experimental import pallas as pl


def kernel(tensor):
    raise NotImplementedError("write your pallas kernel here")


# speedup vs baseline: 3.6226x; 3.6226x over previous
import jax
import jax.numpy as jnp
from jax import lax
from jax.experimental import pallas as pl
from jax.experimental.pallas import tpu as pltpu
from jax.experimental.pallas import tpu_sc as plsc

R = 128
C = 32768
N = R * C
LOG2C = 15
NC = 2
NS = 16
NW = NC * NS
CHUNK = N // NW
SUB = 16384
NSUB = CHUNK // SUB
L = 16
OUTW = 2 * N
PAT = 2048

_MESH = plsc.VectorSubcoreMesh(
    core_axis_name="c", subcore_axis_name="s", num_cores=NC, num_subcores=NS
)

_SEG_SIZES = (16384, 16384, 8192, 4096, 2048, 1024, 512, 256, 128, 64, 32,
              16, 8)
_FILL_SIZES = (1024, 512, 256, 128, 64, 32, 16, 8)


def _wid():
  return lax.axis_index("s") * NC + lax.axis_index("c")


def _pass1_body(in_hbm, stage_hbm, counts_hbm, in_v, pos_v, crow_v):
  wid = _wid()
  iota = lax.iota(jnp.int32, L)
  cvec = jnp.zeros((L,), jnp.int32)
  lp = jnp.full((L,), -1, jnp.int32)

  for si in range(NSUB):
    base = wid * CHUNK + si * SUB
    pltpu.sync_copy(in_hbm.at[pl.ds(base, SUB)], in_v)
    base_iota = base + iota

    def group(g, carry, base_iota=base_iota):
      lc, lp = carry
      for u in range(4):
        off = g * (4 * L) + u * L
        v = in_v[pl.ds(off, L)]
        m = v != 0.0
        pv = base_iota + off
        t = plsc.all_reduce_population_count(m)
        plsc.store_compressed(pos_v.at[pl.ds(lc, L)], pv, mask=m)
        lp = jnp.maximum(lp, jnp.where(m, pv, -1))
        lc = lc + t[0]
      return lc, lp

    lc, lp = lax.fori_loop(0, SUB // (4 * L), group, (jnp.int32(0), lp))
    pltpu.sync_copy(
        pos_v.at[pl.ds(0, SUB)],
        stage_hbm.at[pl.ds((wid * NSUB + si) * SUB, SUB)],
    )
    cvec = jnp.where(iota == si, lc, cvec)

  last = jnp.max(lp)
  cvec = jnp.where(iota == NSUB, last, cvec)
  crow_v[...] = cvec
  pltpu.sync_copy(crow_v, counts_hbm.at[wid])


def _pass2_body(stage_hbm, counts_hbm, out_hbm, cnt_v, pos_v, pair_v, pat_v,
                edge_v, sem):
  wid = _wid()
  iota = lax.iota(jnp.int32, L)
  pltpu.sync_copy(counts_hbm, cnt_v)

  submask = iota < NSUB
  my_base = jnp.int32(0)
  total = jnp.int32(0)
  p_last = jnp.int32(-1)
  myrow = jnp.zeros((L,), jnp.int32)
  for u in range(NW):
    row = cnt_v[u]
    wcnt = jnp.sum(jnp.where(submask, row, 0))
    my_base = my_base + jnp.where(u < wid, wcnt, 0)
    total = total + wcnt
    p_last = jnp.maximum(p_last, row[NSUB])
    myrow = jnp.where(wid == u, row, myrow)

  p_last = jnp.where(total == 0, N - 1, p_last)

  def edge_scatter(src_off, gw_base, nvalid):
    edge_v[...] = pair_v[pl.ds(src_off, L)]
    ok = iota < nvalid
    idx = jnp.where(ok, gw_base + iota, OUTW + iota)
    pltpu.async_copy(edge_v, out_hbm.at[idx], sem).wait()

  goff = my_base
  for si in range(NSUB):
    cseg = myrow[si]
    pltpu.sync_copy(stage_hbm.at[pl.ds((wid * NSUB + si) * SUB, SUB)], pos_v)

    gw = 2 * goff
    a = gw % 8
    words = 2 * cseg
    h = (8 - a) % 8

    def expand(j, _, a=a):
      pv = pos_v[pl.ds(j * L, L)]
      plsc.store_scatter(pair_v, [a + 2 * L * j + 2 * iota], pv >> LOG2C)
      plsc.store_scatter(pair_v, [a + 2 * L * j + 2 * iota + 1], pv & (C - 1))
      return 0

    nv = (cseg + (L - 1)) // L
    lax.fori_loop(0, nv, expand, 0)

    edge_scatter(a, gw, jnp.minimum(h, words))
    tl = (gw + words) % 8
    ts = jnp.maximum(words - tl, h)
    edge_scatter(a + ts, gw + ts, words - ts)
    rem = jnp.maximum(words - tl - h, 0)
    src0 = pl.multiple_of(a + h, 8)
    dst0 = pl.multiple_of(gw + h, 8)
    off = jnp.int32(0)
    for size in _SEG_SIZES:
      take = rem >= size

      @pl.when(take)
      def _(off=off, size=size, src0=src0, dst0=dst0):
        pltpu.sync_copy(
            pair_v.at[pl.ds(pl.multiple_of(src0 + off, 8), size)],
            out_hbm.at[pl.ds(pl.multiple_of(dst0 + off, 8), size)],
        )

      off = off + jnp.where(take, size, 0)
      rem = rem - jnp.where(take, size, 0)
    goff = goff + cseg

  patv = jnp.where(iota % 2 == 0, p_last >> LOG2C, p_last & (C - 1))
  for i in range(PAT // L):
    pat_v[pl.ds(i * L, L)] = patv

  fw = 2 * total
  hw = (8 - fw % 8) % 8

  @pl.when(wid == 0)
  def _():
    edge_v[...] = patv
    ok = iota < jnp.minimum(hw, OUTW - fw)
    idx = jnp.where(ok, fw + iota, OUTW + iota)
    pltpu.async_copy(edge_v, out_hbm.at[idx], sem).wait()

  fb = jnp.minimum(fw + hw, OUTW)
  body = OUTW - fb
  per = ((body // 8 + NW - 1) // NW) * 8
  ws = pl.multiple_of(jnp.minimum(fb + wid * per, OUTW), 8)
  we = jnp.minimum(ws + per, OUTW)
  nwords = we - ws
  npat = nwords // PAT

  def fill(i, _):
    pltpu.sync_copy(
        pat_v.at[pl.ds(0, PAT)],
        out_hbm.at[pl.ds(pl.multiple_of(ws + i * PAT, 8), PAT)],
    )
    return 0

  lax.fori_loop(0, npat, fill, 0)

  rem = nwords - npat * PAT
  off = pl.multiple_of(ws + npat * PAT, 8)
  for size in _FILL_SIZES:
    take = rem >= size

    @pl.when(take)
    def _(off=off, size=size):
      pltpu.sync_copy(
          pat_v.at[pl.ds(0, size)],
          out_hbm.at[pl.ds(pl.multiple_of(off, 8), size)],
      )

    off = off + jnp.where(take, size, 0)
    rem = rem - jnp.where(take, size, 0)


_pass1 = pl.kernel(
    _pass1_body,
    out_type=(
        jax.ShapeDtypeStruct((N,), jnp.int32),
        jax.ShapeDtypeStruct((NW, L), jnp.int32),
    ),
    mesh=_MESH,
    compiler_params=pltpu.CompilerParams(needs_layout_passes=False),
    scratch_types=(
        pltpu.VMEM((SUB,), jnp.float32),
        pltpu.VMEM((SUB + L,), jnp.int32),
        pltpu.VMEM((L,), jnp.int32),
    ),
)

_pass2 = pl.kernel(
    _pass2_body,
    out_type=jax.ShapeDtypeStruct((OUTW + L,), jnp.int32),
    mesh=_MESH,
    compiler_params=pltpu.CompilerParams(needs_layout_passes=False),
    scratch_types=(
        pltpu.VMEM((NW, L), jnp.int32),
        pltpu.VMEM((SUB,), jnp.int32),
        pltpu.VMEM((2 * SUB + 4 * L,), jnp.int32),
        pltpu.VMEM((PAT,), jnp.int32),
        pltpu.VMEM((L,), jnp.int32),
        pltpu.SemaphoreType.DMA,
    ),
)


@jax.jit
def kernel(tensor):
  flat = tensor.reshape(-1)
  stage, counts = _pass1(flat)
  out_flat = _pass2(stage, counts)
  return out_flat[:OUTW].reshape(N, 2).astype(jnp.int64)


# speedup vs baseline: 4.8548x; 1.3401x over previous
import jax
import jax.numpy as jnp
from jax import lax
from jax.experimental import pallas as pl
from jax.experimental.pallas import tpu as pltpu
from jax.experimental.pallas import tpu_sc as plsc

R = 128
C = 32768
N = R * C
LOG2C = 15
NC = 2
NS = 16
NW = NC * NS
CHUNK = N // NW
SUB = 16384
NSUB = CHUNK // SUB
L = 16
OUTW = 2 * N
PAT = 8192

_MESH = plsc.VectorSubcoreMesh(
    core_axis_name="c", subcore_axis_name="s", num_cores=NC, num_subcores=NS
)

_SEG_SIZES = (16384, 16384, 8192, 4096, 2048, 1024, 512, 256, 128, 64, 32,
              16, 8)
_FILL_SIZES = (4096, 2048, 1024, 512, 256, 128, 64, 32, 16, 8)


def _wid():
  return lax.axis_index("s") * NC + lax.axis_index("c")


def _pass1_body(in_hbm, stage_hbm, counts_hbm, in_v0, in_v1, pos_v0, pos_v1,
                crow_v, semi, semo0, semo1):
  wid = _wid()
  iota = lax.iota(jnp.int32, L)
  cvec = jnp.zeros((L,), jnp.int32)
  lp = jnp.full((L,), -1, jnp.int32)
  in_bufs = (in_v0, in_v1)
  pos_bufs = (pos_v0, pos_v1)
  semos = (semo0, semo1)

  def in_src(si):
    return in_hbm.at[pl.ds(wid * CHUNK + si * SUB, SUB)]

  def stage_dst(si):
    return stage_hbm.at[pl.ds((wid * NSUB + si) * SUB, SUB)]

  pltpu.async_copy(in_src(0), in_bufs[0], semi)
  for si in range(NSUB):
    in_v = in_bufs[si % 2]
    pos_v = pos_bufs[si % 2]
    pltpu.make_async_copy(in_src(si), in_v, semi).wait()
    if si + 1 < NSUB:
      pltpu.async_copy(in_src(si + 1), in_bufs[(si + 1) % 2], semi)
    if si >= 2:
      pltpu.make_async_copy(
          pos_v.at[pl.ds(0, SUB)], stage_dst(si - 2), semos[si % 2]
      ).wait()

    base = wid * CHUNK + si * SUB
    base_iota = base + iota

    def group(g, carry, base_iota=base_iota, in_v=in_v, pos_v=pos_v):
      lc, lp = carry
      for u in range(4):
        off = g * (4 * L) + u * L
        v = in_v[pl.ds(off, L)]
        m = v != 0.0
        pv = base_iota + off
        t = plsc.all_reduce_population_count(m)
        plsc.store_compressed(pos_v.at[pl.ds(lc, L)], pv, mask=m)
        lp = jnp.maximum(lp, jnp.where(m, pv, -1))
        lc = lc + t[0]
      return lc, lp

    lc, lp = lax.fori_loop(0, SUB // (4 * L), group, (jnp.int32(0), lp))
    pltpu.async_copy(pos_v.at[pl.ds(0, SUB)], stage_dst(si), semos[si % 2])
    cvec = jnp.where(iota == si, lc, cvec)

  for si in (NSUB - 2, NSUB - 1):
    pltpu.make_async_copy(
        pos_bufs[si % 2].at[pl.ds(0, SUB)], stage_dst(si), semos[si % 2]
    ).wait()

  last = jnp.max(lp)
  cvec = jnp.where(iota == NSUB, last, cvec)
  crow_v[...] = cvec
  pltpu.sync_copy(crow_v, counts_hbm.at[wid])


def _pass2_body(stage_hbm, counts_hbm, out_hbm, cnt_v, pos_v0, pos_v1,
                pair_v0, pair_v1, pat_v, edge_v, semr, semw0, semw1, seme,
                semf):
  wid = _wid()
  iota = lax.iota(jnp.int32, L)
  pltpu.sync_copy(counts_hbm, cnt_v)

  submask = iota < NSUB
  my_base = jnp.int32(0)
  total = jnp.int32(0)
  p_last = jnp.int32(-1)
  myrow = jnp.zeros((L,), jnp.int32)
  for u in range(NW):
    row = cnt_v[u]
    wcnt = jnp.sum(jnp.where(submask, row, 0))
    my_base = my_base + jnp.where(u < wid, wcnt, 0)
    total = total + wcnt
    p_last = jnp.maximum(p_last, row[NSUB])
    myrow = jnp.where(wid == u, row, myrow)

  p_last = jnp.where(total == 0, N - 1, p_last)

  pos_bufs = (pos_v0, pos_v1)
  pair_bufs = (pair_v0, pair_v1)
  semws = (semw0, semw1)

  def stage_src(si):
    return stage_hbm.at[pl.ds((wid * NSUB + si) * SUB, SUB)]

  def edge_scatter(pair_v, src_off, gw_base, nvalid):
    @pl.when(nvalid > 0)
    def _():
      lidx = jnp.minimum(iota, nvalid - 1)
      edge_v[...] = plsc.load_gather(pair_v, [src_off + lidx])
      pltpu.async_copy(edge_v, out_hbm.at[gw_base + lidx], seme).wait()

  def seg_geom(goff, cseg):
    gw = 2 * goff
    a = gw % 8
    words = 2 * cseg
    h = (8 - a) % 8
    tl = (gw + words) % 8
    return gw, a, words, h, tl

  def core_copies(si, goff, cseg, drain):
    pair_v = pair_bufs[si % 2]
    semw = semws[si % 2]
    gw, a, words, h, tl = seg_geom(goff, cseg)
    rem = jnp.maximum(words - tl - h, 0)
    src0 = pl.multiple_of(a + h, 8)
    dst0 = pl.multiple_of(gw + h, 8)
    off = jnp.int32(0)
    for size in _SEG_SIZES:
      take = rem >= size

      @pl.when(take)
      def _(off=off, size=size, src0=src0, dst0=dst0, pair_v=pair_v,
            semw=semw):
        src = pair_v.at[pl.ds(pl.multiple_of(src0 + off, 8), size)]
        dst = out_hbm.at[pl.ds(pl.multiple_of(dst0 + off, 8), size)]
        if drain:
          pltpu.make_async_copy(src, dst, semw).wait()
        else:
          pltpu.async_copy(src, dst, semw)

      off = off + jnp.where(take, size, 0)
      rem = rem - jnp.where(take, size, 0)

  goffs = []
  goff = my_base
  for si in range(NSUB):
    goffs.append(goff)
    goff = goff + myrow[si]

  pltpu.async_copy(stage_src(0), pos_v0, semr)
  for si in range(NSUB):
    pos_v = pos_bufs[si % 2]
    pair_v = pair_bufs[si % 2]
    cseg = myrow[si]
    pltpu.make_async_copy(stage_src(si), pos_v, semr).wait()
    if si + 1 < NSUB:
      pltpu.async_copy(stage_src(si + 1), pos_bufs[(si + 1) % 2], semr)
    if si >= 2:
      core_copies(si - 2, goffs[si - 2], myrow[si - 2], drain=True)

    gw, a, words, h, tl = seg_geom(goffs[si], cseg)

    def expand(j, _, a=a, pos_v=pos_v, pair_v=pair_v):
      pv = pos_v[pl.ds(j * L, L)]
      plsc.store_scatter(pair_v, [a + 2 * L * j + 2 * iota], pv >> LOG2C)
      plsc.store_scatter(pair_v, [a + 2 * L * j + 2 * iota + 1], pv & (C - 1))
      return 0

    nv = (cseg + (L - 1)) // L
    lax.fori_loop(0, nv, expand, 0)

    edge_scatter(pair_v, a, gw, jnp.minimum(h, words))
    ts = jnp.maximum(words - tl, h)
    edge_scatter(pair_v, a + ts, gw + ts, words - ts)
    core_copies(si, goffs[si], cseg, drain=False)

  for si in (NSUB - 2, NSUB - 1):
    core_copies(si, goffs[si], myrow[si], drain=True)

  patv = jnp.where(iota % 2 == 0, p_last >> LOG2C, p_last & (C - 1))

  def build_pat(i, _):
    pat_v[pl.ds(i * L, L)] = patv
    return 0

  lax.fori_loop(0, PAT // L, build_pat, 0)

  fw = 2 * total
  hw = (8 - fw % 8) % 8

  @pl.when((wid == 0) & (hw > 0))
  def _():
    lidx = jnp.minimum(iota, hw - 1)
    edge_v[...] = plsc.load_gather(pat_v, [lidx])
    pltpu.async_copy(edge_v, out_hbm.at[fw + lidx], seme).wait()

  fb = jnp.minimum(fw + hw, OUTW)
  body = OUTW - fb
  per = ((body // 8 + NW - 1) // NW) * 8
  ws = pl.multiple_of(jnp.minimum(fb + wid * per, OUTW), 8)
  we = jnp.minimum(ws + per, OUTW)
  nwords = we - ws
  npat = nwords // PAT

  def fill_issue(i, _):
    pltpu.async_copy(
        pat_v.at[pl.ds(0, PAT)],
        out_hbm.at[pl.ds(pl.multiple_of(ws + i * PAT, 8), PAT)],
        semf,
    )
    return 0

  def fill_drain(i, _):
    pltpu.make_async_copy(
        pat_v.at[pl.ds(0, PAT)],
        out_hbm.at[pl.ds(pl.multiple_of(ws + i * PAT, 8), PAT)],
        semf,
    ).wait()
    return 0

  lax.fori_loop(0, npat, fill_issue, 0)

  rem = nwords - npat * PAT
  off = pl.multiple_of(ws + npat * PAT, 8)
  for drain in (False, True):
    off_i = off
    rem_i = rem
    for size in _FILL_SIZES:
      take = rem_i >= size

      @pl.when(take)
      def _(off_i=off_i, size=size, drain=drain):
        src = pat_v.at[pl.ds(0, size)]
        dst = out_hbm.at[pl.ds(pl.multiple_of(off_i, 8), size)]
        if drain:
          pltpu.make_async_copy(src, dst, semf).wait()
        else:
          pltpu.async_copy(src, dst, semf)

      off_i = off_i + jnp.where(take, size, 0)
      rem_i = rem_i - jnp.where(take, size, 0)
    if not drain:
      lax.fori_loop(0, npat, fill_drain, 0)


_pass1 = pl.kernel(
    _pass1_body,
    out_type=(
        jax.ShapeDtypeStruct((N,), jnp.int32),
        jax.ShapeDtypeStruct((NW, L), jnp.int32),
    ),
    mesh=_MESH,
    compiler_params=pltpu.CompilerParams(needs_layout_passes=False),
    scratch_types=(
        pltpu.VMEM((SUB,), jnp.float32),
        pltpu.VMEM((SUB,), jnp.float32),
        pltpu.VMEM((SUB + L,), jnp.int32),
        pltpu.VMEM((SUB + L,), jnp.int32),
        pltpu.VMEM((L,), jnp.int32),
        pltpu.SemaphoreType.DMA,
        pltpu.SemaphoreType.DMA,
        pltpu.SemaphoreType.DMA,
    ),
)

_pass2 = pl.kernel(
    _pass2_body,
    out_type=jax.ShapeDtypeStruct((OUTW,), jnp.int32),
    mesh=_MESH,
    compiler_params=pltpu.CompilerParams(needs_layout_passes=False),
    scratch_types=(
        pltpu.VMEM((NW, L), jnp.int32),
        pltpu.VMEM((SUB,), jnp.int32),
        pltpu.VMEM((SUB,), jnp.int32),
        pltpu.VMEM((2 * SUB + 4 * L,), jnp.int32),
        pltpu.VMEM((2 * SUB + 4 * L,), jnp.int32),
        pltpu.VMEM((PAT,), jnp.int32),
        pltpu.VMEM((L,), jnp.int32),
        pltpu.SemaphoreType.DMA,
        pltpu.SemaphoreType.DMA,
        pltpu.SemaphoreType.DMA,
        pltpu.SemaphoreType.DMA,
        pltpu.SemaphoreType.DMA,
    ),
)


@jax.jit
def kernel(tensor):
  flat = tensor.reshape(-1)
  stage, counts = _pass1(flat)
  out_flat = _pass2(stage, counts)
  return out_flat.reshape(N, 2).astype(jnp.int64)


# speedup vs baseline: 59.7598x; 12.3095x over previous
import jax
import jax.numpy as jnp
from jax import lax
from jax.experimental import pallas as pl
from jax.experimental.pallas import tpu as pltpu
from jax.experimental.pallas import tpu_sc as plsc

R = 128
C = 32768
N = R * C
LOG2C = 15
NC = 2
NS = 16
NW = NC * NS
CHUNK = N // NW
SUB = 16384
NSUB = CHUNK // SUB
L = 16
PATW = 2048

_MESH = plsc.VectorSubcoreMesh(
    core_axis_name="c", subcore_axis_name="s", num_cores=NC, num_subcores=NS
)

_SEG_SIZES = (16384, 8192, 4096, 2048, 1024, 512, 256, 128, 64, 32, 16, 8)
_FILL_SIZES = (1024, 512, 256, 128, 64, 32, 16, 8)


def _wid():
  return lax.axis_index("s") * NC + lax.axis_index("c")


def _pass1_body(in_hbm, stage_hbm, counts_hbm, in_v0, in_v1, pos_v0, pos_v1,
                crow_v, semi, semo0, semo1):
  wid = _wid()
  iota = lax.iota(jnp.int32, L)
  cvec = jnp.zeros((L,), jnp.int32)
  lp = jnp.full((L,), -1, jnp.int32)
  in_bufs = (in_v0, in_v1)
  pos_bufs = (pos_v0, pos_v1)
  semos = (semo0, semo1)

  def in_src(si):
    return in_hbm.at[pl.ds(wid * CHUNK + si * SUB, SUB)]

  def stage_dst(si):
    return stage_hbm.at[pl.ds((wid * NSUB + si) * SUB, SUB)]

  pltpu.async_copy(in_src(0), in_bufs[0], semi)
  for si in range(NSUB):
    in_v = in_bufs[si % 2]
    pos_v = pos_bufs[si % 2]
    pltpu.make_async_copy(in_src(si), in_v, semi).wait()
    if si + 1 < NSUB:
      pltpu.async_copy(in_src(si + 1), in_bufs[(si + 1) % 2], semi)
    if si >= 2:
      pltpu.make_async_copy(
          pos_v.at[pl.ds(0, SUB)], stage_dst(si - 2), semos[si % 2]
      ).wait()

    base = wid * CHUNK + si * SUB
    base_iota = base + iota

    def group(g, carry, base_iota=base_iota, in_v=in_v, pos_v=pos_v):
      lc, lp = carry
      for u in range(4):
        off = g * (4 * L) + u * L
        v = in_v[pl.ds(off, L)]
        m = v != 0.0
        pv = base_iota + off
        t = plsc.all_reduce_population_count(m)
        plsc.store_compressed(pos_v.at[pl.ds(lc, L)], pv, mask=m)
        lp = jnp.maximum(lp, jnp.where(m, pv, -1))
        lc = lc + t[0]
      return lc, lp

    lc, lp = lax.fori_loop(0, SUB // (4 * L), group, (jnp.int32(0), lp))
    pltpu.async_copy(pos_v.at[pl.ds(0, SUB)], stage_dst(si), semos[si % 2])
    cvec = jnp.where(iota == si, lc, cvec)

  for si in (NSUB - 2, NSUB - 1):
    pltpu.make_async_copy(
        pos_bufs[si % 2].at[pl.ds(0, SUB)], stage_dst(si), semos[si % 2]
    ).wait()

  last = jnp.max(lp)
  cvec = jnp.where(iota == NSUB, last, cvec)
  crow_v[...] = cvec
  pltpu.sync_copy(crow_v, counts_hbm.at[wid])


def _pass2_body(stage_hbm, counts_hbm, rows_hbm, cols_hbm, cnt_v, pos_v0,
                pos_v1, rbuf0, rbuf1, cbuf0, cbuf1, patr_v, patc_v, edge_v,
                semr, semw0, semw1, seme, semf):
  wid = _wid()
  iota = lax.iota(jnp.int32, L)
  pltpu.sync_copy(counts_hbm, cnt_v)

  submask = iota < NSUB
  my_base = jnp.int32(0)
  total = jnp.int32(0)
  p_last = jnp.int32(-1)
  myrow = jnp.zeros((L,), jnp.int32)
  for u in range(NW):
    row = cnt_v[u]
    wcnt = jnp.sum(jnp.where(submask, row, 0))
    my_base = my_base + jnp.where(u < wid, wcnt, 0)
    total = total + wcnt
    p_last = jnp.maximum(p_last, row[NSUB])
    myrow = jnp.where(wid == u, row, myrow)

  p_last = jnp.where(total == 0, N - 1, p_last)

  pos_bufs = (pos_v0, pos_v1)
  rbufs = (rbuf0, rbuf1)
  cbufs = (cbuf0, cbuf1)
  semws = (semw0, semw1)

  def stage_src(si):
    return stage_hbm.at[pl.ds((wid * NSUB + si) * SUB, SUB)]

  def edge_scatter(buf, hbm, src_off, base, nvalid):
    lidx = jnp.minimum(iota, nvalid - 1)
    edge_v[...] = plsc.load_gather(buf, [src_off + lidx])
    pltpu.async_copy(edge_v, hbm.at[base + lidx], seme).wait()

  def core_copies(si, goff, cseg, drain):
    rbuf = rbufs[si % 2]
    cbuf = cbufs[si % 2]
    semw = semws[si % 2]
    a = goff % 8
    h = (8 - a) % 8
    tl = (goff + cseg) % 8
    rem = jnp.maximum(cseg - tl - h, 0)
    src0 = pl.multiple_of(a + h, 8)
    dst0 = pl.multiple_of(goff + h, 8)
    off = jnp.int32(0)
    for size in _SEG_SIZES:
      take = rem >= size

      @pl.when(take)
      def _(off=off, size=size, src0=src0, dst0=dst0, rbuf=rbuf, cbuf=cbuf,
            semw=semw):
        s0 = pl.multiple_of(src0 + off, 8)
        d0 = pl.multiple_of(dst0 + off, 8)
        if drain:
          pltpu.make_async_copy(
              rbuf.at[pl.ds(s0, size)], rows_hbm.at[pl.ds(d0, size)], semw
          ).wait()
          pltpu.make_async_copy(
              cbuf.at[pl.ds(s0, size)], cols_hbm.at[pl.ds(d0, size)], semw
          ).wait()
        else:
          pltpu.async_copy(
              rbuf.at[pl.ds(s0, size)], rows_hbm.at[pl.ds(d0, size)], semw
          )
          pltpu.async_copy(
              cbuf.at[pl.ds(s0, size)], cols_hbm.at[pl.ds(d0, size)], semw
          )

      off = off + jnp.where(take, size, 0)
      rem = rem - jnp.where(take, size, 0)

  goffs = []
  goff = my_base
  for si in range(NSUB):
    goffs.append(goff)
    goff = goff + myrow[si]

  pltpu.async_copy(stage_src(0), pos_v0, semr)
  for si in range(NSUB):
    pos_v = pos_bufs[si % 2]
    rbuf = rbufs[si % 2]
    cbuf = cbufs[si % 2]
    cseg = myrow[si]
    pltpu.make_async_copy(stage_src(si), pos_v, semr).wait()
    if si + 1 < NSUB:
      pltpu.async_copy(stage_src(si + 1), pos_bufs[(si + 1) % 2], semr)
    if si >= 2:
      core_copies(si - 2, goffs[si - 2], myrow[si - 2], drain=True)

    goff = goffs[si]
    a = goff % 8
    h = (8 - a) % 8
    tl = (goff + cseg) % 8

    def expand(j, _, a=a, pos_v=pos_v, rbuf=rbuf, cbuf=cbuf):
      pv = pos_v[pl.ds(j * L, L)]
      rbuf[pl.ds(a + L * j, L)] = pv >> LOG2C
      cbuf[pl.ds(a + L * j, L)] = pv & (C - 1)
      return 0

    nv = (cseg + (L - 1)) // L
    lax.fori_loop(0, nv, expand, 0)

    hn = jnp.minimum(h, cseg)

    @pl.when(hn > 0)
    def _(a=a, goff=goff, hn=hn, rbuf=rbuf, cbuf=cbuf):
      edge_scatter(rbuf, rows_hbm, a, goff, hn)
      edge_scatter(cbuf, cols_hbm, a, goff, hn)

    ts = jnp.maximum(cseg - tl, h)
    tn = cseg - ts

    @pl.when(tn > 0)
    def _(a=a, goff=goff, ts=ts, tn=tn, rbuf=rbuf, cbuf=cbuf):
      edge_scatter(rbuf, rows_hbm, a + ts, goff + ts, tn)
      edge_scatter(cbuf, cols_hbm, a + ts, goff + ts, tn)

    core_copies(si, goffs[si], cseg, drain=False)

  for si in (NSUB - 2, NSUB - 1):
    core_copies(si, goffs[si], myrow[si], drain=True)

  r_last = p_last >> LOG2C
  c_last = p_last & (C - 1)
  rsplat = jnp.zeros((L,), jnp.int32) + r_last
  csplat = jnp.zeros((L,), jnp.int32) + c_last

  def build_pat(i, _):
    patr_v[pl.ds(i * L, L)] = rsplat
    patc_v[pl.ds(i * L, L)] = csplat
    return 0

  lax.fori_loop(0, PATW // L, build_pat, 0)

  hw = (8 - total % 8) % 8

  @pl.when((wid == 0) & (hw > 0))
  def _():
    lidx = jnp.minimum(iota, hw - 1)
    edge_v[...] = rsplat
    pltpu.async_copy(edge_v, rows_hbm.at[total + lidx], seme).wait()
    edge_v[...] = csplat
    pltpu.async_copy(edge_v, cols_hbm.at[total + lidx], seme).wait()

  fb = jnp.minimum(total + hw, N)
  body = N - fb
  per = ((body // 8 + NW - 1) // NW) * 8
  ws = pl.multiple_of(jnp.minimum(fb + wid * per, N), 8)
  we = jnp.minimum(ws + per, N)
  nwords = we - ws
  npat = nwords // PATW

  def fill_issue(i, _):
    d0 = pl.multiple_of(ws + i * PATW, 8)
    pltpu.async_copy(
        patr_v.at[pl.ds(0, PATW)], rows_hbm.at[pl.ds(d0, PATW)], semf
    )
    pltpu.async_copy(
        patc_v.at[pl.ds(0, PATW)], cols_hbm.at[pl.ds(d0, PATW)], semf
    )
    return 0

  def fill_drain(i, _):
    d0 = pl.multiple_of(ws + i * PATW, 8)
    pltpu.make_async_copy(
        patr_v.at[pl.ds(0, PATW)], rows_hbm.at[pl.ds(d0, PATW)], semf
    ).wait()
    pltpu.make_async_copy(
        patc_v.at[pl.ds(0, PATW)], cols_hbm.at[pl.ds(d0, PATW)], semf
    ).wait()
    return 0

  lax.fori_loop(0, npat, fill_issue, 0)

  rem = nwords - npat * PATW
  off = pl.multiple_of(ws + npat * PATW, 8)
  for drain in (False, True):
    off_i = off
    rem_i = rem
    for size in _FILL_SIZES:
      take = rem_i >= size

      @pl.when(take)
      def _(off_i=off_i, size=size, drain=drain):
        d0 = pl.multiple_of(off_i, 8)
        if drain:
          pltpu.make_async_copy(
              patr_v.at[pl.ds(0, size)], rows_hbm.at[pl.ds(d0, size)], semf
          ).wait()
          pltpu.make_async_copy(
              patc_v.at[pl.ds(0, size)], cols_hbm.at[pl.ds(d0, size)], semf
          ).wait()
        else:
          pltpu.async_copy(
              patr_v.at[pl.ds(0, size)], rows_hbm.at[pl.ds(d0, size)], semf
          )
          pltpu.async_copy(
              patc_v.at[pl.ds(0, size)], cols_hbm.at[pl.ds(d0, size)], semf
          )

      off_i = off_i + jnp.where(take, size, 0)
      rem_i = rem_i - jnp.where(take, size, 0)
    if not drain:
      lax.fori_loop(0, npat, fill_drain, 0)


_pass1 = pl.kernel(
    _pass1_body,
    out_type=(
        jax.ShapeDtypeStruct((N,), jnp.int32),
        jax.ShapeDtypeStruct((NW, L), jnp.int32),
    ),
    mesh=_MESH,
    compiler_params=pltpu.CompilerParams(needs_layout_passes=False),
    scratch_types=(
        pltpu.VMEM((SUB,), jnp.float32),
        pltpu.VMEM((SUB,), jnp.float32),
        pltpu.VMEM((SUB + L,), jnp.int32),
        pltpu.VMEM((SUB + L,), jnp.int32),
        pltpu.VMEM((L,), jnp.int32),
        pltpu.SemaphoreType.DMA,
        pltpu.SemaphoreType.DMA,
        pltpu.SemaphoreType.DMA,
    ),
)

_pass2 = pl.kernel(
    _pass2_body,
    out_type=(
        jax.ShapeDtypeStruct((N,), jnp.int32),
        jax.ShapeDtypeStruct((N,), jnp.int32),
    ),
    mesh=_MESH,
    compiler_params=pltpu.CompilerParams(needs_layout_passes=False),
    scratch_types=(
        pltpu.VMEM((NW, L), jnp.int32),
        pltpu.VMEM((SUB,), jnp.int32),
        pltpu.VMEM((SUB,), jnp.int32),
        pltpu.VMEM((SUB + 2 * L,), jnp.int32),
        pltpu.VMEM((SUB + 2 * L,), jnp.int32),
        pltpu.VMEM((SUB + 2 * L,), jnp.int32),
        pltpu.VMEM((SUB + 2 * L,), jnp.int32),
        pltpu.VMEM((PATW,), jnp.int32),
        pltpu.VMEM((PATW,), jnp.int32),
        pltpu.VMEM((L,), jnp.int32),
        pltpu.SemaphoreType.DMA,
        pltpu.SemaphoreType.DMA,
        pltpu.SemaphoreType.DMA,
        pltpu.SemaphoreType.DMA,
        pltpu.SemaphoreType.DMA,
    ),
)


@jax.jit
def kernel(tensor):
  flat = tensor.reshape(-1)
  stage, counts = _pass1(flat)
  rows, cols = _pass2(stage, counts)
  return jnp.stack([rows, cols], axis=1).astype(jnp.int64)
